# baseline (speedup 1.0000x reference)
import jax
import jax.numpy as jnp
from jax import lax
from jax.experimental import pallas as pl
from jax.experimental.pallas import tpu as pltpu

B = 16
N = 8192
M = 1024


def _fps_kernel(x_ref, y_ref, z_ref, p_ref, c_ref):
    x = x_ref[...]
    y = y_ref[...]
    z = z_ref[...]
    p = p_ref[...]
    iota = lax.broadcasted_iota(jnp.int32, (B, N), 1)
    one = jnp.ones((B, N), dtype=jnp.float32)
    zerow = jnp.zeros((B, N), dtype=jnp.float32)
    lane = lax.broadcasted_iota(jnp.int32, (B, 3 * B), 1)
    row = lax.broadcasted_iota(jnp.int32, (B, 3 * B), 0)
    zeros = jnp.zeros((B, 3 * B), dtype=jnp.float32)

    def body(k, carry):
        d_prev, fx, fy, fz, out_row = carry
        c_ref[pl.ds(k, 1), :] = out_row

        dx = x - fx
        dy = y - fy
        dz = z - fz
        dist = (dx * dx + dz * dz) + dy * dy
        d = jnp.minimum(d_prev, dist)

        m = jnp.max(d, axis=1, keepdims=True)
        cand = jnp.where(d == m, iota, N)
        j = jnp.min(cand, axis=1, keepdims=True)
        onehot = jnp.where(iota == j, one, zerow)
        prod = lax.dot_general(
            onehot,
            p,
            (((1,), (0,)), ((), ())),
            precision=lax.Precision.HIGHEST,
            preferred_element_type=jnp.float32,
        )
        nfx = jnp.sum(jnp.where(lane == row, prod, zeros), axis=1, keepdims=True)
        nfy = jnp.sum(jnp.where(lane == row + B, prod, zeros), axis=1, keepdims=True)
        nfz = jnp.sum(jnp.where(lane == row + 2 * B, prod, zeros), axis=1, keepdims=True)
        diag = (lane == row) | (lane == row + B) | (lane == row + 2 * B)
        nrow = jnp.sum(jnp.where(diag, prod, zeros), axis=0, keepdims=True)
        return d, nfx, nfy, nfz, nrow

    init = (
        jnp.full((B, N), jnp.inf, dtype=jnp.float32),
        x[:, 0:1],
        y[:, 0:1],
        z[:, 0:1],
        jnp.concatenate(
            [x[:, 0:1].reshape(1, B), y[:, 0:1].reshape(1, B), z[:, 0:1].reshape(1, B)],
            axis=1,
        ),
    )
    lax.fori_loop(0, M, body, init)


@jax.jit
def kernel(xyz):
    x = xyz[:, :, 0]
    y = xyz[:, :, 1]
    z = xyz[:, :, 2]
    p = xyz.transpose(1, 2, 0).reshape(N, 3 * B)
    c = pl.pallas_call(
        _fps_kernel,
        out_shape=jax.ShapeDtypeStruct((M, 3 * B), jnp.float32),
    )(x, y, z, p)
    return jnp.stack([c[:, :B].T, c[:, B : 2 * B].T, c[:, 2 * B :].T], axis=-1)


# speedup vs baseline: 3.3788x; 3.3788x over previous
import jax
import jax.numpy as jnp
from jax import lax
from jax.experimental import pallas as pl
from jax.experimental.pallas import tpu as pltpu

B = 16
N = 8192
M = 1024
CH = 512
NC = N // CH


def _fps_kernel(x_ref, y_ref, z_ref, c_ref, d_ref):
    d_ref[...] = jnp.full((B, N), jnp.inf, dtype=jnp.float32)
    iota_c = lax.broadcasted_iota(jnp.int32, (B, CH), 1)
    zero_c = jnp.zeros((B, CH), dtype=jnp.float32)
    zero_s = jnp.zeros((B, NC), dtype=jnp.float32)

    def body(k, carry):
        fx, fy, fz, out_row = carry
        c_ref[pl.ds(k, 1), :] = out_row

        ms, js, fxs, fys, fzs = [], [], [], [], []
        for c in range(NC):
            sl = pl.ds(c * CH, CH)
            xc = x_ref[:, sl]
            yc = y_ref[:, sl]
            zc = z_ref[:, sl]
            dx = xc - fx
            dy = yc - fy
            dz = zc - fz
            dist = (dx * dx + dz * dz) + dy * dy
            dc = jnp.minimum(d_ref[:, sl], dist)
            d_ref[:, sl] = dc

            idx = iota_c + c * CH
            m_c = jnp.max(dc, axis=1, keepdims=True)
            cand = jnp.where(dc == m_c, idx, N)
            j_c = jnp.min(cand, axis=1, keepdims=True)
            oh = idx == j_c
            fx_c = jnp.sum(jnp.where(oh, xc, zero_c), axis=1, keepdims=True)
            fy_c = jnp.sum(jnp.where(oh, yc, zero_c), axis=1, keepdims=True)
            fz_c = jnp.sum(jnp.where(oh, zc, zero_c), axis=1, keepdims=True)
            ms.append(m_c)
            js.append(j_c)
            fxs.append(fx_c)
            fys.append(fy_c)
            fzs.append(fz_c)

        m_all = jnp.concatenate(ms, axis=1)
        j_all = jnp.concatenate(js, axis=1)
        m = jnp.max(m_all, axis=1, keepdims=True)
        j = jnp.min(jnp.where(m_all == m, j_all, N), axis=1, keepdims=True)
        ohc = j_all == j
        fx_all = jnp.concatenate(fxs, axis=1)
        fy_all = jnp.concatenate(fys, axis=1)
        fz_all = jnp.concatenate(fzs, axis=1)
        nfx = jnp.sum(jnp.where(ohc, fx_all, zero_s), axis=1, keepdims=True)
        nfy = jnp.sum(jnp.where(ohc, fy_all, zero_s), axis=1, keepdims=True)
        nfz = jnp.sum(jnp.where(ohc, fz_all, zero_s), axis=1, keepdims=True)
        nrow = jnp.concatenate(
            [nfx.reshape(1, B), nfy.reshape(1, B), nfz.reshape(1, B)], axis=1
        )
        return nfx, nfy, nfz, nrow

    x0 = x_ref[:, 0:1]
    y0 = y_ref[:, 0:1]
    z0 = z_ref[:, 0:1]
    init = (
        x0,
        y0,
        z0,
        jnp.concatenate(
            [x0.reshape(1, B), y0.reshape(1, B), z0.reshape(1, B)], axis=1
        ),
    )
    lax.fori_loop(0, M, body, init)


@jax.jit
def kernel(xyz):
    x = xyz[:, :, 0]
    y = xyz[:, :, 1]
    z = xyz[:, :, 2]
    c = pl.pallas_call(
        _fps_kernel,
        out_shape=jax.ShapeDtypeStruct((M, 3 * B), jnp.float32),
        scratch_shapes=[pltpu.VMEM((B, N), jnp.float32)],
    )(x, y, z)
    return jnp.stack([c[:, :B].T, c[:, B : 2 * B].T, c[:, 2 * B :].T], axis=-1)
